# baseline (speedup 1.0000x reference)
import functools

import jax
import jax.numpy as jnp
from jax import lax
from jax.experimental import pallas as pl
from jax.experimental.pallas import tpu as pltpu
from jax.experimental.pallas import tpu_sc as plsc

NUM_OCTETS = 4
EMB = 32
VOCAB = 256
LANES = 16
NUM_CORES = 2
NUM_SUBCORES = 16
NUM_WORKERS = NUM_CORES * NUM_SUBCORES
IDX_CHUNK = 128


def _fold_tables_body(tables_ref, wt_ref, bias_ref, out_ref):
    w0 = wt_ref[0]
    w1 = wt_ref[1]
    w2 = wt_ref[2]
    m_first = (w0 + w1) * 0.25
    m_mid = (w0 + w1 + w2) * 0.25
    m_last = (w1 + w2) * 0.25
    b4 = bias_ref[...] * 0.25
    mats = (m_first, m_mid, m_mid, m_last)
    for j in range(NUM_OCTETS):
        prod = jnp.dot(tables_ref[j], mats[j], preferred_element_type=jnp.float32)
        out_ref[j * VOCAB:(j + 1) * VOCAB, :] = prod + b4


def _fold_tables(tables, conv_w, conv_b):
    wt = jnp.transpose(conv_w, (2, 1, 0))
    bias_row = conv_b.reshape(1, EMB)
    return pl.pallas_call(
        _fold_tables_body,
        out_shape=jax.ShapeDtypeStruct((NUM_OCTETS * VOCAB, EMB), jnp.float32),
    )(tables, wt, bias_row)


def _make_sc_lookup(batch):
    bpw = batch // NUM_WORKERS
    xpw = bpw * NUM_OCTETS
    jb = bpw // LANES
    n_chunks = bpw // IDX_CHUNK
    mesh = plsc.VectorSubcoreMesh(core_axis_name="c", subcore_axis_name="s")

    @functools.partial(
        pl.kernel,
        mesh=mesh,
        out_type=jax.ShapeDtypeStruct((batch // 4, 4 * EMB), jnp.float32),
        scratch_types=[
            pltpu.VMEM((xpw,), jnp.int32),
            pltpu.VMEM((xpw,), jnp.int32),
            pltpu.VMEM((bpw, EMB), jnp.float32),
            pltpu.VMEM((bpw // 4, 4 * EMB), jnp.float32),
            pltpu.SemaphoreType.DMA,
        ],
        compiler_params=pltpu.CompilerParams(
            use_tc_tiling_on_sc=False, needs_layout_passes=False),
    )
    def sc_lookup(table_hbm, xt_hbm, out_hbm, xv, idx_v, slab, slab_w, sem):
        wid = lax.axis_index("s") * NUM_CORES + lax.axis_index("c")
        base = wid * bpw

        for j in range(NUM_OCTETS):
            pltpu.sync_copy(xt_hbm.at[j, pl.ds(base, bpw)],
                            xv.at[pl.ds(j * bpw, bpw)])

        def build_octet(j):
            for blk in range(jb):
                sl = pl.ds(j * bpw + blk * LANES, LANES)
                idx_v[sl] = xv[sl] + j * VOCAB

        def fire_octet(j, add):
            return [
                pltpu.async_copy(
                    table_hbm.at[idx_v.at[pl.ds(j * bpw + c * IDX_CHUNK, IDX_CHUNK)]],
                    slab.at[pl.ds(c * IDX_CHUNK, IDX_CHUNK)],
                    sem,
                    add=add,
                )
                for c in range(n_chunks)
            ]

        build_octet(0)
        first = fire_octet(0, add=False)
        for j in range(1, NUM_OCTETS):
            build_octet(j)
        for cp in first:
            cp.wait()
        rest = []
        for j in range(1, NUM_OCTETS):
            rest.extend(fire_octet(j, add=True))
        for cp in rest:
            cp.wait()

        def widen_row(p, carry):
            for q in range(4):
                for col in range(0, EMB, LANES):
                    slab_w[p, pl.ds(q * EMB + col, LANES)] = \
                        slab[4 * p + q, pl.ds(col, LANES)]
            return carry
        lax.fori_loop(0, bpw // 4, widen_row, 0)
        pltpu.sync_copy(slab_w, out_hbm.at[pl.ds(wid * (bpw // 4), bpw // 4)])

    return sc_lookup


def _unpack_body(in_ref, out_ref):
    wide = in_ref[...]
    parts = [wide[:, q * EMB:(q + 1) * EMB] for q in range(4)]
    stacked = jnp.stack(parts, axis=1)
    out_ref[...] = stacked.reshape(out_ref.shape)


def _unpack_rows(packed, batch):
    grid = 16
    rows_in = packed.shape[0] // grid
    return pl.pallas_call(
        _unpack_body,
        grid=(grid,),
        in_specs=[pl.BlockSpec((rows_in, 4 * EMB), lambda i: (i, 0))],
        out_specs=pl.BlockSpec((rows_in * 4, EMB), lambda i: (i, 0)),
        out_shape=jax.ShapeDtypeStruct((batch, EMB), jnp.float32),
    )(packed)


def kernel(x, tables, conv_w, conv_b):
    batch = x.shape[0]
    table = _fold_tables(tables, conv_w, conv_b)
    xt = x.astype(jnp.int32).T
    packed = _make_sc_lookup(batch)(table, xt)
    return _unpack_rows(packed, batch)


# speedup vs baseline: 1.2293x; 1.2293x over previous
import functools

import jax
import jax.numpy as jnp
from jax import lax
from jax.experimental import pallas as pl
from jax.experimental.pallas import tpu as pltpu
from jax.experimental.pallas import tpu_sc as plsc

NUM_OCTETS = 4
EMB = 32
VOCAB = 256
LANES = 16
NUM_CORES = 2
NUM_SUBCORES = 16
NUM_WORKERS = NUM_CORES * NUM_SUBCORES
IDX_CHUNK = 128


def _fold_tables_body(tables_ref, wt_ref, bias_ref, out_ref):
    w0 = wt_ref[0]
    w1 = wt_ref[1]
    w2 = wt_ref[2]
    m_first = (w0 + w1) * 0.25
    m_mid = (w0 + w1 + w2) * 0.25
    m_last = (w1 + w2) * 0.25
    b4 = bias_ref[...] * 0.25
    mats = (m_first, m_mid, m_mid, m_last)
    for j in range(NUM_OCTETS):
        prod = jnp.dot(tables_ref[j], mats[j], preferred_element_type=jnp.float32)
        out_ref[j * VOCAB:(j + 1) * VOCAB, :] = prod + b4


def _fold_tables(tables, conv_w, conv_b):
    wt = jnp.transpose(conv_w, (2, 1, 0))
    bias_row = conv_b.reshape(1, EMB)
    return pl.pallas_call(
        _fold_tables_body,
        out_shape=jax.ShapeDtypeStruct((NUM_OCTETS * VOCAB, EMB), jnp.float32),
    )(tables, wt, bias_row)


def _make_sc_lookup(batch):
    bpw = batch // NUM_WORKERS
    xpw = bpw * NUM_OCTETS
    jb = bpw // LANES
    n_chunks = bpw // IDX_CHUNK
    mesh = plsc.VectorSubcoreMesh(core_axis_name="c", subcore_axis_name="s")

    @functools.partial(
        pl.kernel,
        mesh=mesh,
        out_type=jax.ShapeDtypeStruct((batch, EMB), jnp.float32),
        scratch_types=[
            pltpu.VMEM((xpw,), jnp.int32),
            pltpu.VMEM((xpw,), jnp.int32),
            pltpu.VMEM((bpw, EMB), jnp.float32),
            pltpu.SemaphoreType.DMA,
            pltpu.SemaphoreType.DMA,
        ],
        compiler_params=pltpu.CompilerParams(
            use_tc_tiling_on_sc=False, needs_layout_passes=False),
    )
    def sc_lookup(table_hbm, xt_hbm, out_hbm, xv, idx_v, slab, sem, sem2):
        wid = lax.axis_index("s") * NUM_CORES + lax.axis_index("c")
        base = wid * bpw

        for j in range(NUM_OCTETS):
            pltpu.sync_copy(xt_hbm.at[j, pl.ds(base, bpw)],
                            xv.at[pl.ds(j * bpw, bpw)])

        def build_octet(j):
            for blk in range(jb):
                sl = pl.ds(j * bpw + blk * LANES, LANES)
                idx_v[sl] = xv[sl] + j * VOCAB

        def fire(j, c, s, add):
            return pltpu.async_copy(
                table_hbm.at[idx_v.at[pl.ds(j * bpw + c * IDX_CHUNK, IDX_CHUNK)]],
                slab.at[pl.ds(c * IDX_CHUNK, IDX_CHUNK)],
                s,
                add=add,
            )

        build_octet(0)
        first = [fire(0, c, sem, add=False) for c in range(n_chunks)]
        for j in range(1, NUM_OCTETS):
            build_octet(j)
        rest = []
        for c in range(n_chunks):
            first[c].wait()
            for j in range(1, NUM_OCTETS):
                rest.append(fire(j, c, sem2, add=True))
        for cp in rest:
            cp.wait()

        pltpu.sync_copy(slab, out_hbm.at[pl.ds(base, bpw)])

    return sc_lookup


def kernel(x, tables, conv_w, conv_b):
    batch = x.shape[0]
    table = _fold_tables(tables, conv_w, conv_b)
    xt = x.astype(jnp.int32).T
    return _make_sc_lookup(batch)(table, xt)
